# Initial kernel scaffold; baseline (speedup 1.0000x reference)
import functools

import jax
import jax.numpy as jnp
from jax import lax
from jax.experimental import pallas as pl
from jax.experimental.pallas import tpu as pltpu
from jax.experimental.pallas import tpu_sc as plsc

N_NODES = 10000
D_FEAT = 128
N_EDGES = 320000

NC = 2
NS = 16
NW = NC * NS
EPW = N_EDGES // NW
K = 80
NCH = EPW // K
RPT = N_NODES // NS


def _sc_scatter_body(x_hbm, src_hbm, dst_hbm, ew_hbm, acc_hbm,
                     acc_sh, src_v, dst_v, ew_v, rows_v, sem):
    c = lax.axis_index("c")
    s = lax.axis_index("s")
    gid = c * NS + s

    pltpu.sync_copy(src_hbm.at[gid], src_v)
    pltpu.sync_copy(dst_hbm.at[gid], dst_v)
    pltpu.sync_copy(ew_hbm.at[pl.ds(gid * EPW, EPW)], ew_v)

    def _zero(i, carry):
        rows_v[i // 8, pl.ds((i % 8) * 16, 16)] = jnp.zeros((16,), jnp.float32)
        return carry
    lax.fori_loop(0, K * 8, _zero, 0)
    for j in range(RPT // K):
        pltpu.sync_copy(rows_v, acc_sh.at[pl.ds(s * RPT + j * K, K)])
    rem = RPT % K
    if rem:
        pltpu.sync_copy(rows_v.at[pl.ds(0, rem)],
                        acc_sh.at[pl.ds(s * RPT + (RPT // K) * K, rem)])
    plsc.subcore_barrier()

    def _chunk(k, carry):
        pltpu.async_copy(x_hbm.at[src_v.at[k]], rows_v, sem).wait()

        def _scale(r, inner):
            w16 = plsc.load_gather(
                ew_v, [jnp.full((16,), k * K + r, jnp.int32)])
            for cc in range(8):
                sl = pl.ds(cc * 16, 16)
                rows_v[r, sl] = rows_v[r, sl] * w16
            return inner
        lax.fori_loop(0, K, _scale, 0)

        pltpu.sync_copy(rows_v, acc_sh.at[dst_v.at[k]], add=True)
        return carry
    lax.fori_loop(0, NCH, _chunk, 0)

    plsc.subcore_barrier()
    pltpu.sync_copy(acc_sh.at[pl.ds(s * RPT, RPT)],
                    acc_hbm.at[c, pl.ds(s * RPT, RPT)])


_sc_scatter = functools.partial(
    pl.kernel,
    out_type=jax.ShapeDtypeStruct((NC, N_NODES, D_FEAT), jnp.float32),
    mesh=plsc.VectorSubcoreMesh(core_axis_name="c", subcore_axis_name="s"),
    scratch_types=[
        pltpu.VMEM_SHARED((N_NODES, D_FEAT), jnp.float32),
        pltpu.VMEM((NCH, K), jnp.int32),
        pltpu.VMEM((NCH, K), jnp.int32),
        pltpu.VMEM((EPW,), jnp.float32),
        pltpu.VMEM((K, D_FEAT), jnp.float32),
        pltpu.SemaphoreType.DMA,
    ],
)(_sc_scatter_body)


def _finish_body(acc_ref, o_ref):
    t = acc_ref[0] + acc_ref[1]
    t = jnp.maximum(t, 0.0)
    nrm = jnp.sqrt(jnp.sum(t * t, axis=1, keepdims=True))
    o_ref[...] = t / jnp.maximum(nrm, 1e-12)


_ROWS_PER_BLK = 1000


def _finish(acc):
    return pl.pallas_call(
        _finish_body,
        grid=(N_NODES // _ROWS_PER_BLK,),
        in_specs=[pl.BlockSpec((NC, _ROWS_PER_BLK, D_FEAT),
                               lambda i: (0, i, 0))],
        out_specs=pl.BlockSpec((_ROWS_PER_BLK, D_FEAT), lambda i: (i, 0)),
        out_shape=jax.ShapeDtypeStruct((N_NODES, D_FEAT), jnp.float32),
    )(acc)


def kernel(x, edge, edge_weight):
    src = edge[0].reshape(NW, NCH, K)
    dst = edge[2].reshape(NW, NCH, K)
    acc = _sc_scatter(x, src, dst, edge_weight)
    return _finish(acc)


# speedup vs baseline: 13.6459x; 13.6459x over previous
import functools

import jax
import jax.numpy as jnp
from jax import lax
from jax.experimental import pallas as pl
from jax.experimental.pallas import tpu as pltpu
from jax.experimental.pallas import tpu_sc as plsc

N_NODES = 10000
D_FEAT = 128
N_EDGES = 320000

NC = 2
NS = 16
NW = NC * NS
EPW = N_EDGES // NW
K = 80
NB = 5
CB = 25
N_PAD = 10240
RPT = N_PAD // NS


def _sc_scatter_body(x_hbm, src_hbm, dst_hbm, ew_hbm, acc_hbm,
                     acc_sh, src_v, dst_v, ew_v, rows_v, sem):
    c = lax.axis_index("c")
    s = lax.axis_index("s")
    gid = c * NS + s

    def _zero(i, carry):
        rows_v[i // 8, pl.ds((i % 8) * 16, 16)] = jnp.zeros((16,), jnp.float32)
        return carry
    lax.fori_loop(0, K * 8, _zero, 0)
    for j in range(RPT // K):
        pltpu.sync_copy(rows_v, acc_sh.at[pl.ds(s * RPT + j * K, K)])
    plsc.subcore_barrier()

    def _block(b, carry):
        pltpu.sync_copy(src_hbm.at[gid, b], src_v)
        pltpu.sync_copy(dst_hbm.at[gid, b], dst_v)
        pltpu.sync_copy(ew_hbm.at[gid, b], ew_v)

        def _chunk(k, c2):
            pltpu.async_copy(x_hbm.at[src_v.at[k]], rows_v, sem).wait()

            def _scale(g, inner):
                w_win = ew_v[pl.ds(k * K + g * 16, 16)]
                for r16 in range(16):
                    r = g * 16 + r16
                    w16 = jnp.broadcast_to(w_win[r16], (16,))
                    for cc in range(8):
                        sl = pl.ds(cc * 16, 16)
                        rows_v[r, sl] = rows_v[r, sl] * w16
                return inner
            lax.fori_loop(0, K // 16, _scale, 0)

            pltpu.sync_copy(rows_v, acc_sh.at[dst_v.at[k]], add=True)
            return c2
        lax.fori_loop(0, CB, _chunk, 0)
        return carry
    lax.fori_loop(0, NB, _block, 0)

    plsc.subcore_barrier()
    pltpu.sync_copy(acc_sh.at[pl.ds(s * RPT, RPT)],
                    acc_hbm.at[c, pl.ds(s * RPT, RPT)])


_sc_scatter = functools.partial(
    pl.kernel,
    out_type=jax.ShapeDtypeStruct((NC, N_PAD, D_FEAT), jnp.float32),
    mesh=plsc.VectorSubcoreMesh(core_axis_name="c", subcore_axis_name="s"),
    scratch_types=[
        pltpu.VMEM_SHARED((N_PAD, D_FEAT), jnp.float32),
        pltpu.VMEM((CB, K), jnp.int32),
        pltpu.VMEM((CB, K), jnp.int32),
        pltpu.VMEM((CB * K,), jnp.float32),
        pltpu.VMEM((K, D_FEAT), jnp.float32),
        pltpu.SemaphoreType.DMA,
    ],
)(_sc_scatter_body)


def _finish_body(acc_ref, o_ref):
    t = acc_ref[0] + acc_ref[1]
    t = jnp.maximum(t, 0.0)
    nrm = jnp.sqrt(jnp.sum(t * t, axis=1, keepdims=True))
    o_ref[...] = t / jnp.maximum(nrm, 1e-12)


_ROWS_PER_BLK = 1024


def _finish(acc):
    return pl.pallas_call(
        _finish_body,
        grid=(N_PAD // _ROWS_PER_BLK,),
        in_specs=[pl.BlockSpec((NC, _ROWS_PER_BLK, D_FEAT),
                               lambda i: (0, i, 0))],
        out_specs=pl.BlockSpec((_ROWS_PER_BLK, D_FEAT), lambda i: (i, 0)),
        out_shape=jax.ShapeDtypeStruct((N_PAD, D_FEAT), jnp.float32),
    )(acc)


def kernel(x, edge, edge_weight):
    src = edge[0].reshape(NW, NB, CB, K)
    dst = edge[2].reshape(NW, NB, CB, K)
    ew = edge_weight.reshape(NW, NB, CB * K)
    acc = _sc_scatter(x, src, dst, ew)
    return _finish(acc)[:N_NODES]


# speedup vs baseline: 20.8657x; 1.5291x over previous
import functools

import jax
import jax.numpy as jnp
from jax import lax
from jax.experimental import pallas as pl
from jax.experimental.pallas import tpu as pltpu
from jax.experimental.pallas import tpu_sc as plsc

N_NODES = 10000
D_FEAT = 128
N_EDGES = 320000

NC = 2
NS = 16
NW = NC * NS
EPW = N_EDGES // NW
K = 80
NB = 5
CB = 25
N_PAD = 10240
RPT = N_PAD // NS


def _sc_scatter_body(x_hbm, src_hbm, dst_hbm, ew_hbm, acc_hbm,
                     acc_sh, src_v, dst_v, ew_v, rows_a, rows_b,
                     sem_a, sem_b):
    c = lax.axis_index("c")
    s = lax.axis_index("s")
    gid = c * NS + s

    def _zero(i, carry):
        rows_a[i // 8, pl.ds((i % 8) * 16, 16)] = jnp.zeros((16,), jnp.float32)
        return carry
    lax.fori_loop(0, K * 8, _zero, 0)
    for j in range(RPT // K):
        pltpu.sync_copy(rows_a, acc_sh.at[pl.ds(s * RPT + j * K, K)])
    plsc.subcore_barrier()

    def _scale(buf, base, g, inner):
        w_win = ew_v[pl.ds(base + g * 16, 16)]
        for r16 in range(16):
            r = g * 16 + r16
            w16 = jnp.broadcast_to(w_win[r16], (16,))
            for cc in range(8):
                sl = pl.ds(cc * 16, 16)
                buf[r, sl] = buf[r, sl] * w16
        return inner

    def _process(buf, k):
        lax.fori_loop(0, K // 16,
                      functools.partial(_scale, buf, k * K), 0)
        pltpu.sync_copy(buf, acc_sh.at[dst_v.at[k]], add=True)

    def _block(b, carry):
        pltpu.sync_copy(src_hbm.at[gid, b], src_v)
        pltpu.sync_copy(dst_hbm.at[gid, b], dst_v)
        pltpu.sync_copy(ew_hbm.at[gid, b], ew_v)

        cp_a = pltpu.async_copy(x_hbm.at[src_v.at[0]], rows_a, sem_a)

        def _pair(p, c2):
            k0 = 2 * p
            pltpu.async_copy(x_hbm.at[src_v.at[k0 + 1]], rows_b, sem_b)
            pltpu.make_async_copy(x_hbm.at[src_v.at[k0]], rows_a,
                                  sem_a).wait()
            _process(rows_a, k0)
            pltpu.async_copy(x_hbm.at[src_v.at[k0 + 2]], rows_a, sem_a)
            pltpu.make_async_copy(x_hbm.at[src_v.at[k0 + 1]], rows_b,
                                  sem_b).wait()
            _process(rows_b, k0 + 1)
            return c2
        lax.fori_loop(0, CB // 2, _pair, 0)

        del cp_a
        pltpu.make_async_copy(x_hbm.at[src_v.at[CB - 1]], rows_a,
                              sem_a).wait()
        _process(rows_a, CB - 1)
        return carry
    lax.fori_loop(0, NB, _block, 0)

    plsc.subcore_barrier()
    pltpu.sync_copy(acc_sh.at[pl.ds(s * RPT, RPT)],
                    acc_hbm.at[c, pl.ds(s * RPT, RPT)])


_sc_scatter = functools.partial(
    pl.kernel,
    out_type=jax.ShapeDtypeStruct((NC, N_PAD, D_FEAT), jnp.float32),
    mesh=plsc.VectorSubcoreMesh(core_axis_name="c", subcore_axis_name="s"),
    scratch_types=[
        pltpu.VMEM_SHARED((N_PAD, D_FEAT), jnp.float32),
        pltpu.VMEM((CB, K), jnp.int32),
        pltpu.VMEM((CB, K), jnp.int32),
        pltpu.VMEM((CB * K,), jnp.float32),
        pltpu.VMEM((K, D_FEAT), jnp.float32),
        pltpu.VMEM((K, D_FEAT), jnp.float32),
        pltpu.SemaphoreType.DMA,
        pltpu.SemaphoreType.DMA,
    ],
)(_sc_scatter_body)


def _finish_body(acc_ref, o_ref):
    t = acc_ref[0] + acc_ref[1]
    t = jnp.maximum(t, 0.0)
    nrm = jnp.sqrt(jnp.sum(t * t, axis=1, keepdims=True))
    o_ref[...] = t / jnp.maximum(nrm, 1e-12)


_ROWS_PER_BLK = 1024


def _finish(acc):
    return pl.pallas_call(
        _finish_body,
        grid=(N_PAD // _ROWS_PER_BLK,),
        in_specs=[pl.BlockSpec((NC, _ROWS_PER_BLK, D_FEAT),
                               lambda i: (0, i, 0))],
        out_specs=pl.BlockSpec((_ROWS_PER_BLK, D_FEAT), lambda i: (i, 0)),
        out_shape=jax.ShapeDtypeStruct((N_PAD, D_FEAT), jnp.float32),
    )(acc)


def kernel(x, edge, edge_weight):
    src = edge[0].reshape(NW, NB, CB, K)
    dst = edge[2].reshape(NW, NB, CB, K)
    ew = edge_weight.reshape(NW, NB, CB * K)
    acc = _sc_scatter(x, src, dst, ew)
    return _finish(acc)[:N_NODES]
